# baseline (speedup 1.0000x reference)
import jax
import jax.numpy as jnp
from jax import lax
from jax.experimental import pallas as pl
from jax.experimental.pallas import tpu as pltpu
from jax.experimental.pallas import tpu_sc as plsc

B, S, D = 16, 2048, 128
T = 1024
NC, NS = 2, 16
L = 16
NPASS = 2
BATCH_PER_PASS = B // (NC * NPASS)
CH = 128
ACC_ROWS = BATCH_PER_PASS * T
SUB_PER_TILE = BATCH_PER_PASS * S // NS
OUT_PER_TILE = ACC_ROWS // NS


def _body(hid, idx, ones_in, out, acc, cnt, idxb, datab, cntb):
    c = lax.axis_index("c")
    w = lax.axis_index("s")

    for p in range(NPASS):
        def _zero_datab(i, _):
            datab[i // (D // L), pl.ds((i % (D // L)) * L, L)] = jnp.zeros((L,), jnp.float32)
            return 0
        lax.fori_loop(0, CH * (D // L), _zero_datab, 0)

        for r in range(OUT_PER_TILE // CH):
            pltpu.sync_copy(datab, acc.at[pl.ds(w * OUT_PER_TILE + r * CH, CH)])
            pltpu.sync_copy(datab, cnt.at[pl.ds(w * OUT_PER_TILE + r * CH, CH)])
        pltpu.sync_copy(ones_in, cntb)
        plsc.subcore_barrier()

        b = c * (NPASS * BATCH_PER_PASS) + p * BATCH_PER_PASS + w // 4
        base = b * S + (w % 4) * SUB_PER_TILE
        tok_off = (w // 4) * T
        for k in range(SUB_PER_TILE // CH):
            off = base + k * CH
            pltpu.sync_copy(idx.at[pl.ds(off, CH)], idxb)

            def _offset(i, _):
                idxb[pl.ds(i * L, L)] = idxb[pl.ds(i * L, L)] + tok_off
                return 0
            lax.fori_loop(0, CH // L, _offset, 0)

            pltpu.sync_copy(hid.at[pl.ds(off, CH)], datab)
            pltpu.sync_copy(datab, acc.at[idxb], add=True)
            pltpu.sync_copy(cntb, cnt.at[idxb], add=True)
        plsc.subcore_barrier()

        out_base = (c * NPASS + p) * BATCH_PER_PASS * T
        for k in range(OUT_PER_TILE // CH):
            r0 = w * OUT_PER_TILE + k * CH
            pltpu.sync_copy(acc.at[pl.ds(r0, CH)], datab)
            pltpu.sync_copy(cnt.at[pl.ds(r0, CH)], cntb)

            def _row(i, _):
                for j in range(D // L):
                    datab[i, pl.ds(j * L, L)] = datab[i, pl.ds(j * L, L)] / jnp.maximum(
                        cntb[i, pl.ds(j * L, L)], 1.0)
                return 0
            lax.fori_loop(0, CH, _row, 0)

            pltpu.sync_copy(datab, out.at[pl.ds(out_base + r0, CH)])
        plsc.subcore_barrier()


@jax.jit
def _aggregate(hidden, idx32):
    mesh = plsc.VectorSubcoreMesh(
        core_axis_name="c", subcore_axis_name="s", num_cores=NC, num_subcores=NS
    )
    out = pl.kernel(
        _body,
        out_type=jax.ShapeDtypeStruct((B * T, D), jnp.float32),
        mesh=mesh,
        scratch_types=[
            pltpu.VMEM_SHARED((ACC_ROWS, D), jnp.float32),
            pltpu.VMEM_SHARED((ACC_ROWS, D), jnp.float32),
            pltpu.VMEM((CH,), jnp.int32),
            pltpu.VMEM((CH, D), jnp.float32),
            pltpu.VMEM((CH, D), jnp.float32),
        ],
    )(hidden.reshape(B * S, D), idx32.reshape(B * S),
      jnp.ones((CH, D), jnp.float32))
    return out.reshape(B, T, D)


def kernel(hidden, ori_indexes):
    return _aggregate(hidden, ori_indexes.astype(jnp.int32))
